# baseline (speedup 1.0000x reference)
import functools

import jax
import jax.numpy as jnp
from jax.experimental import pallas as pl
from jax.experimental.pallas import tpu as pltpu

B, N, P = 32, 512, 4
E, G, H = 4, 16, 64
N_PROCESS = 3
BPB = 4


def _encoder_kernel(a_ref, x_ref, wg0cat_ref, wnem_ref, wnex_ref, bne_ref,
                    wg1blk_ref, wne1m_ref, wne1h_ref, wne1x_ref, bne1_ref,
                    enc_ref):
    f32 = jnp.float32
    bf16 = jnp.bfloat16
    for i in range(BPB):
        a = a_ref[i].reshape(E * N, N)
        x = x_ref[i]

        z0 = jnp.dot(x, wg0cat_ref[...], preferred_element_type=f32)
        y0 = jnp.dot(a, z0, preferred_element_type=f32,
                     precision=jax.lax.Precision.DEFAULT)
        msgs0 = jnp.concatenate(
            [y0[e * N:(e + 1) * N, e * G:(e + 1) * G] for e in range(E)],
            axis=1)
        msgs0 = jax.nn.relu(msgs0)
        h0 = jax.nn.relu(
            jnp.dot(msgs0, wnem_ref[...], preferred_element_type=f32)
            + jnp.dot(x, wnex_ref[...], preferred_element_type=f32)
            + bne_ref[...])

        y1 = jnp.dot(a, h0, preferred_element_type=f32,
                     precision=jax.lax.Precision.DEFAULT)
        y1cat = jnp.concatenate(
            [y1[e * N:(e + 1) * N, :] for e in range(E)], axis=1)
        msgs1 = jax.nn.relu(
            jnp.dot(y1cat, wg1blk_ref[...], preferred_element_type=f32))
        enc_ref[i] = jax.nn.relu(
            jnp.dot(msgs1, wne1m_ref[...], preferred_element_type=f32)
            + jnp.dot(h0, wne1h_ref[...], preferred_element_type=f32)
            + jnp.dot(x, wne1x_ref[...], preferred_element_type=f32)
            + bne1_ref[...])


def _glimpse_kernel(enc_ref, vec_ref, wq_ref, bq_ref, wreft_ref, bref_ref,
                    wfc1_ref, wfc2_ref, out_ref):
    f32 = jnp.float32
    BG = B // 2
    enc = enc_ref[...]
    enc2 = enc.reshape(BG * N, H)
    query = jnp.mean(enc, axis=1)
    u2 = (jnp.dot(enc2, wreft_ref[...], preferred_element_type=f32)
          + bref_ref[...]).reshape(BG, N, H)
    node_id = jax.lax.broadcasted_iota(jnp.int32, (BG, N), 1)
    valid = node_id < (N - 2)
    for _ in range(N_PROCESS):
        u1 = jnp.dot(query, wq_ref[...], preferred_element_type=f32) + bq_ref[...]
        t = jnp.tanh(u2 + u1[:, None, :])
        u = jnp.dot(t.reshape(BG * N, H), vec_ref[...],
                    preferred_element_type=f32).reshape(BG, N)
        u = jnp.where(valid, u, -jnp.inf)
        u = u - jnp.max(u, axis=1, keepdims=True)
        ex = jnp.where(valid, jnp.exp(u), 0.0)
        attn = ex / jnp.sum(ex, axis=1, keepdims=True)
        query = jnp.sum(attn[:, :, None] * enc, axis=1)

    hid = jax.nn.relu(jnp.dot(query, wfc1_ref[...], preferred_element_type=f32))
    pred = jnp.dot(hid, wfc2_ref[...], preferred_element_type=f32)
    out_ref[...] = jnp.broadcast_to(pred, (B // 2, 128))


@jax.jit
def kernel(node_features, heterogeneous_edges, W_g0, W_ne, b_ne, W_g1, W_ne1,
           b_ne1, Vec, W_q, b_q, W_ref, b_ref, W_fc1, W_fc2):
    wg0cat = jnp.transpose(W_g0, (1, 0, 2)).reshape(P, E * G)
    wg1blk = jax.scipy.linalg.block_diag(*[W_g1[e] for e in range(E)])
    wne_m = W_ne[:E * G]
    wne_x = W_ne[E * G:]
    wne1_m = W_ne1[:E * G]
    wne1_h = W_ne1[E * G:E * G + H]
    wne1_x = W_ne1[E * G + H:]

    full = lambda *shape: pl.BlockSpec(shape, lambda b: (0,) * len(shape))
    enc = pl.pallas_call(
        _encoder_kernel,
        grid=(B // BPB,),
        in_specs=[
            pl.BlockSpec((BPB, E, N, N), lambda b: (b, 0, 0, 0)),
            pl.BlockSpec((BPB, N, P), lambda b: (b, 0, 0)),
            full(P, E * G),
            full(E * G, H),
            full(P, H),
            full(1, H),
            full(E * H, E * G),
            full(E * G, H),
            full(H, H),
            full(P, H),
            full(1, H),
        ],
        out_specs=pl.BlockSpec((BPB, N, H), lambda b: (b, 0, 0)),
        out_shape=jax.ShapeDtypeStruct((B, N, H), jnp.float32),
        compiler_params=pltpu.CompilerParams(
            dimension_semantics=("parallel",),
            vmem_limit_bytes=100 * 1024 * 1024),
    )(heterogeneous_edges, node_features, wg0cat, wne_m, wne_x,
      b_ne.reshape(1, H), wg1blk, wne1_m, wne1_h, wne1_x, b_ne1.reshape(1, H))

    out = pl.pallas_call(
        _glimpse_kernel,
        grid=(2,),
        in_specs=[
            pl.BlockSpec((B // 2, N, H), lambda b: (b, 0, 0)),
            full(H, 1),
            full(H, H),
            full(1, H),
            full(H, H),
            full(1, H),
            full(H, H),
            full(H, 1),
        ],
        out_specs=pl.BlockSpec((B // 2, 128), lambda b: (b, 0)),
        out_shape=jax.ShapeDtypeStruct((B, 128), jnp.float32),
        compiler_params=pltpu.CompilerParams(
            dimension_semantics=("parallel",)),
    )(enc, Vec.reshape(H, 1), W_q, b_q.reshape(1, H), W_ref.T,
      b_ref.reshape(1, H), W_fc1, W_fc2)
    return out[:, 0]


# speedup vs baseline: 1.0440x; 1.0440x over previous
import functools

import jax
import jax.numpy as jnp
from jax.experimental import pallas as pl
from jax.experimental.pallas import tpu as pltpu

B, N, P = 32, 512, 4
E, G, H = 4, 16, 64
N_PROCESS = 3
BPB = 2


def _encoder_kernel(a_ref, x_ref, wg0cat_ref, wnem_ref, wnex_ref, bne_ref,
                    wg1blk_ref, wne1m_ref, wne1h_ref, wne1x_ref, bne1_ref,
                    enc_ref):
    f32 = jnp.float32
    bf16 = jnp.bfloat16
    for i in range(BPB):
        a = a_ref[i].reshape(E * N, N)
        x = x_ref[i]

        z0 = jnp.dot(x, wg0cat_ref[...], preferred_element_type=f32)
        y0 = jnp.dot(a, z0, preferred_element_type=f32,
                     precision=jax.lax.Precision.DEFAULT)
        msgs0 = jnp.concatenate(
            [y0[e * N:(e + 1) * N, e * G:(e + 1) * G] for e in range(E)],
            axis=1)
        msgs0 = jax.nn.relu(msgs0)
        h0 = jax.nn.relu(
            jnp.dot(msgs0, wnem_ref[...], preferred_element_type=f32)
            + jnp.dot(x, wnex_ref[...], preferred_element_type=f32)
            + bne_ref[...])

        y1 = jnp.dot(a, h0, preferred_element_type=f32,
                     precision=jax.lax.Precision.DEFAULT)
        y1cat = jnp.concatenate(
            [y1[e * N:(e + 1) * N, :] for e in range(E)], axis=1)
        msgs1 = jax.nn.relu(
            jnp.dot(y1cat, wg1blk_ref[...], preferred_element_type=f32))
        enc_ref[i] = jax.nn.relu(
            jnp.dot(msgs1, wne1m_ref[...], preferred_element_type=f32)
            + jnp.dot(h0, wne1h_ref[...], preferred_element_type=f32)
            + jnp.dot(x, wne1x_ref[...], preferred_element_type=f32)
            + bne1_ref[...])


def _glimpse_kernel(enc_ref, vec_ref, wq_ref, bq_ref, wreft_ref, bref_ref,
                    wfc1_ref, wfc2_ref, out_ref):
    f32 = jnp.float32
    BG = B // 2
    enc = enc_ref[...]
    enc_t = jnp.transpose(enc, (0, 2, 1))
    u2 = (jnp.dot(enc.reshape(BG * N, H), wreft_ref[...],
                  preferred_element_type=f32)
          + bref_ref[...]).reshape(BG, N, H)
    u2_t = jnp.transpose(u2, (0, 2, 1))
    query = jnp.mean(enc_t, axis=2)
    vec_b = vec_ref[...][None, :, :]
    node_id = jax.lax.broadcasted_iota(jnp.int32, (BG, N), 1)
    valid = node_id < (N - 2)
    for _ in range(N_PROCESS):
        u1 = jnp.dot(query, wq_ref[...], preferred_element_type=f32) + bq_ref[...]
        t = jnp.tanh(u2_t + u1[:, :, None])
        u = jnp.sum(t * vec_b, axis=1)
        u = jnp.where(valid, u, -jnp.inf)
        u = u - jnp.max(u, axis=1, keepdims=True)
        ex = jnp.where(valid, jnp.exp(u), 0.0)
        attn = ex / jnp.sum(ex, axis=1, keepdims=True)
        query = jnp.sum(enc_t * attn[:, None, :], axis=2)

    hid = jax.nn.relu(jnp.dot(query, wfc1_ref[...], preferred_element_type=f32))
    pred = jnp.dot(hid, wfc2_ref[...], preferred_element_type=f32)
    out_ref[...] = jnp.broadcast_to(pred, (B // 2, 128))


@jax.jit
def kernel(node_features, heterogeneous_edges, W_g0, W_ne, b_ne, W_g1, W_ne1,
           b_ne1, Vec, W_q, b_q, W_ref, b_ref, W_fc1, W_fc2):
    wg0cat = jnp.transpose(W_g0, (1, 0, 2)).reshape(P, E * G)
    wg1blk = jax.scipy.linalg.block_diag(*[W_g1[e] for e in range(E)])
    wne_m = W_ne[:E * G]
    wne_x = W_ne[E * G:]
    wne1_m = W_ne1[:E * G]
    wne1_h = W_ne1[E * G:E * G + H]
    wne1_x = W_ne1[E * G + H:]

    full = lambda *shape: pl.BlockSpec(shape, lambda b: (0,) * len(shape))
    enc = pl.pallas_call(
        _encoder_kernel,
        grid=(B // BPB,),
        in_specs=[
            pl.BlockSpec((BPB, E, N, N), lambda b: (b, 0, 0, 0)),
            pl.BlockSpec((BPB, N, P), lambda b: (b, 0, 0)),
            full(P, E * G),
            full(E * G, H),
            full(P, H),
            full(1, H),
            full(E * H, E * G),
            full(E * G, H),
            full(H, H),
            full(P, H),
            full(1, H),
        ],
        out_specs=pl.BlockSpec((BPB, N, H), lambda b: (b, 0, 0)),
        out_shape=jax.ShapeDtypeStruct((B, N, H), jnp.float32),
        compiler_params=pltpu.CompilerParams(
            dimension_semantics=("parallel",),
            vmem_limit_bytes=100 * 1024 * 1024),
    )(heterogeneous_edges, node_features, wg0cat, wne_m, wne_x,
      b_ne.reshape(1, H), wg1blk, wne1_m, wne1_h, wne1_x, b_ne1.reshape(1, H))

    out = pl.pallas_call(
        _glimpse_kernel,
        grid=(2,),
        in_specs=[
            pl.BlockSpec((B // 2, N, H), lambda b: (b, 0, 0)),
            full(H, 1),
            full(H, H),
            full(1, H),
            full(H, H),
            full(1, H),
            full(H, H),
            full(H, 1),
        ],
        out_specs=pl.BlockSpec((B // 2, 128), lambda b: (b, 0)),
        out_shape=jax.ShapeDtypeStruct((B, 128), jnp.float32),
        compiler_params=pltpu.CompilerParams(
            dimension_semantics=("parallel",)),
    )(enc, Vec.reshape(H, 1), W_q, b_q.reshape(1, H), W_ref.T,
      b_ref.reshape(1, H), W_fc1, W_fc2)
    return out[:, 0]
